# baseline (speedup 1.0000x reference)
import functools

import jax
import jax.numpy as jnp
from jax import lax
from jax.experimental import pallas as pl
from jax.experimental.pallas import tpu as pltpu
from jax.experimental.pallas import tpu_sc as plsc

_VOCAB = 100000
_D = 128
_B = 16384
_C = 5
_CTX = 2 * _C
_NPOS = _B * _CTX

_CHUNK = 128
_GRP = 2
_GROWS = _GRP * _CHUNK
_NBUF = 3


def _sc_info():
    try:
        info = plsc.get_sparse_core_info()
        return info.num_cores, info.num_subcores
    except Exception:
        return 2, 16


def _sc_gather(inp_idx, pos_idx, neg_idx, in_embed, out_embed, nc, ns):
    nw = nc * ns
    inp_per_w = _B // nw
    ctx_per_w = _NPOS // nw
    inp_chunks = inp_per_w // _CHUNK
    ctx_chunks = ctx_per_w // _CHUNK
    inp_groups = inp_chunks // _GRP
    ctx_groups = ctx_chunks // _GRP

    mesh = plsc.VectorSubcoreMesh(core_axis_name="c", subcore_axis_name="s")

    @functools.partial(
        pl.kernel,
        mesh=mesh,
        out_type=[
            jax.ShapeDtypeStruct((_B, _D), jnp.float32),
            jax.ShapeDtypeStruct((_NPOS, _D), jnp.float32),
            jax.ShapeDtypeStruct((_NPOS, _D), jnp.float32),
        ],
        scratch_types=[
            pltpu.VMEM((inp_chunks, _CHUNK), jnp.int32),
            pltpu.VMEM((ctx_chunks, _CHUNK), jnp.int32),
            pltpu.VMEM((ctx_chunks, _CHUNK), jnp.int32),
            pltpu.VMEM((_NBUF, _GROWS, _D), jnp.float32),
            pltpu.SemaphoreType.DMA,
            pltpu.SemaphoreType.DMA,
            pltpu.SemaphoreType.DMA,
            pltpu.SemaphoreType.DMA,
        ],
    )
    def gather_kernel(inp_idx_hbm, pos_idx_hbm, neg_idx_hbm, in_hbm, out_hbm,
                      o_inp, o_pos, o_neg, inp_iv, pos_iv, neg_iv, rows_v,
                      gsem, w0, w1, w2):
        wid = lax.axis_index("s") * nc + lax.axis_index("c")
        wsems = (w0, w1, w2)

        pltpu.sync_copy(inp_idx_hbm.at[wid], inp_iv)
        pltpu.sync_copy(pos_idx_hbm.at[wid], pos_iv)
        pltpu.sync_copy(neg_idx_hbm.at[wid], neg_iv)

        def run_job(table, idx_v, ngroups, nbuf, out, rows_per_w):
            base = wid * rows_per_w
            nfull = ngroups // nbuf
            rem = ngroups - nfull * nbuf

            def fire(g, b):
                cps = []
                for s in range(_GRP):
                    cps.append(pltpu.async_copy(
                        table.at[idx_v.at[g * _GRP + s]],
                        rows_v.at[b, pl.ds(s * _CHUNK, _CHUNK)],
                        gsem))
                return cps

            def drain(g, b, cps):
                for cp in cps:
                    cp.wait()
                pltpu.async_copy(
                    rows_v.at[b],
                    out.at[pl.ds(base + g * _GROWS, _GROWS)],
                    wsems[b])

            def wait_write(b):
                pltpu.make_async_copy(
                    rows_v.at[b],
                    out.at[pl.ds(base, _GROWS)],
                    wsems[b]).wait()

            def body(i, carry):
                cps = []
                for b in range(nbuf):
                    @pl.when(i > 0)
                    def _w(b=b):
                        wait_write(b)
                    cps.append(fire(i * nbuf + b, b))
                for b in range(nbuf):
                    drain(i * nbuf + b, b, cps[b])
                return carry

            lax.fori_loop(0, nfull, body, 0)

            tail = []
            for b in range(rem):
                g = nfull * nbuf + b
                wait_write(b)
                tail.append(fire(g, b))
            for b in range(rem):
                drain(nfull * nbuf + b, b, tail[b])
            for b in range(nbuf):
                wait_write(b)

        run_job(in_hbm, inp_iv, inp_groups, 2, o_inp, inp_per_w)
        run_job(out_hbm, pos_iv, ctx_groups, _NBUF, o_pos, ctx_per_w)
        run_job(out_hbm, neg_iv, ctx_groups, _NBUF, o_neg, ctx_per_w)

    return gather_kernel(inp_idx, pos_idx, neg_idx, in_embed, out_embed)


_BLK = 512
_GRID = _B // _BLK


def _loss_body(pos_ref, neg_ref, inp_ref, w_ref, b_ref, out_ref, acc_ref):
    i = pl.program_id(0)
    a = inp_ref[...]
    an = jnp.sqrt(jnp.sum(a * a, axis=1))

    def proj(ref):
        t = jnp.zeros((_BLK, _D), jnp.float32) + b_ref[...]
        for c in range(_CTX):
            t = t + jnp.dot(ref[c], w_ref[c],
                            preferred_element_type=jnp.float32)
        return t

    def cosv(x):
        num = jnp.sum(a * x, axis=1)
        den = jnp.maximum(an * jnp.sqrt(jnp.sum(x * x, axis=1)), 1e-8)
        return num / den

    c = (1.0 - cosv(proj(pos_ref))) + jnp.maximum(cosv(proj(neg_ref)), 0.0)

    @pl.when(i == 0)
    def _init():
        acc_ref[...] = jnp.zeros_like(acc_ref)

    acc_ref[...] += c.reshape(8, _BLK // 8)

    @pl.when(i == _GRID - 1)
    def _fin():
        loss = jnp.sum(acc_ref[...]) * (1.0 / _B)
        out_ref[...] = loss + jnp.zeros((8, _BLK // 8), jnp.float32)


def _tc_loss(pos3, neg3, inp_rows, w1r, b1):
    return pl.pallas_call(
        _loss_body,
        grid=(_GRID,),
        in_specs=[
            pl.BlockSpec((_CTX, _BLK, _D), lambda i: (0, i, 0)),
            pl.BlockSpec((_CTX, _BLK, _D), lambda i: (0, i, 0)),
            pl.BlockSpec((_BLK, _D), lambda i: (i, 0)),
            pl.BlockSpec((_CTX, _D, _D), lambda i: (0, 0, 0)),
            pl.BlockSpec((1, _D), lambda i: (0, 0)),
        ],
        out_specs=pl.BlockSpec((8, _BLK // 8), lambda i: (0, 0)),
        out_shape=jax.ShapeDtypeStruct((8, _BLK // 8), jnp.float32),
        scratch_shapes=[pltpu.VMEM((8, _BLK // 8), jnp.float32)],
    )(pos3, neg3, inp_rows, w1r, b1)


def kernel(input_labels, pos_labels, neg_labels, in_embed, out_embed, W1, b1):
    nc, ns = _sc_info()
    nw = nc * ns
    inp_idx = input_labels.astype(jnp.int32).reshape(nw, -1, _CHUNK)
    pos_idx = pos_labels.astype(jnp.int32).T.reshape(nw, -1, _CHUNK)
    neg_idx = neg_labels.astype(jnp.int32).T.reshape(nw, -1, _CHUNK)

    inp_rows, pos_rows, neg_rows = _sc_gather(
        inp_idx, pos_idx, neg_idx, in_embed, out_embed, nc, ns)

    w1r = W1.reshape(_D, _CTX, _D).transpose(1, 2, 0)

    acc = _tc_loss(
        pos_rows.reshape(_CTX, _B, _D),
        neg_rows.reshape(_CTX, _B, _D),
        inp_rows,
        w1r,
        b1.reshape(1, _D),
    )
    return acc[0, 0]


# speedup vs baseline: 1.0398x; 1.0398x over previous
import functools

import jax
import jax.numpy as jnp
from jax import lax
from jax.experimental import pallas as pl
from jax.experimental.pallas import tpu as pltpu
from jax.experimental.pallas import tpu_sc as plsc

_VOCAB = 100000
_D = 128
_B = 16384
_C = 5
_CTX = 2 * _C
_NPOS = _B * _CTX

_CHUNK = 128
_GRP = 2
_GROWS = _GRP * _CHUNK
_NBUF = 3


def _sc_info():
    try:
        info = plsc.get_sparse_core_info()
        return info.num_cores, info.num_subcores
    except Exception:
        return 2, 16


def _sc_gather(inp_idx, pos_idx, neg_idx, in_embed, out_embed, nc, ns):
    nw = nc * ns
    inp_per_w = _B // nw
    ctx_per_w = _NPOS // nw
    inp_chunks = inp_per_w // _CHUNK
    ctx_chunks = ctx_per_w // _CHUNK
    inp_groups = inp_chunks // _GRP
    ctx_groups = ctx_chunks // _GRP

    mesh = plsc.VectorSubcoreMesh(core_axis_name="c", subcore_axis_name="s")

    @functools.partial(
        pl.kernel,
        mesh=mesh,
        out_type=[
            jax.ShapeDtypeStruct((_B, _D), jnp.float32),
            jax.ShapeDtypeStruct((_NPOS, _D), jnp.float32),
            jax.ShapeDtypeStruct((_NPOS, _D), jnp.float32),
        ],
        scratch_types=[
            pltpu.VMEM((inp_chunks, _CHUNK), jnp.int32),
            pltpu.VMEM((ctx_chunks, _CHUNK), jnp.int32),
            pltpu.VMEM((ctx_chunks, _CHUNK), jnp.int32),
            pltpu.VMEM((_NBUF, _GROWS, _D), jnp.float32),
            pltpu.SemaphoreType.DMA,
            pltpu.SemaphoreType.DMA,
            pltpu.SemaphoreType.DMA,
            pltpu.SemaphoreType.DMA,
        ],
    )
    def gather_kernel(inp_idx_hbm, pos_idx_hbm, neg_idx_hbm, in_hbm, out_hbm,
                      o_inp, o_pos, o_neg, inp_iv, pos_iv, neg_iv, rows_v,
                      gsem, w0, w1, w2):
        wid = lax.axis_index("s") * nc + lax.axis_index("c")
        wsems = (w0, w1, w2)

        pltpu.sync_copy(inp_idx_hbm.at[wid], inp_iv)
        pltpu.sync_copy(pos_idx_hbm.at[wid], pos_iv)
        pltpu.sync_copy(neg_idx_hbm.at[wid], neg_iv)

        def run_job(table, idx_v, ngroups, nbuf, out, rows_per_w):
            base = wid * rows_per_w
            nfull = ngroups // nbuf
            rem = ngroups - nfull * nbuf

            def fire(g, b):
                cps = []
                for s in range(_GRP):
                    cps.append(pltpu.async_copy(
                        table.at[idx_v.at[g * _GRP + s]],
                        rows_v.at[b, pl.ds(s * _CHUNK, _CHUNK)],
                        gsem))
                return cps

            def drain(g, b, cps):
                for cp in cps:
                    cp.wait()
                pltpu.async_copy(
                    rows_v.at[b],
                    out.at[pl.ds(base + g * _GROWS, _GROWS)],
                    wsems[b])

            def wait_write(b):
                pltpu.make_async_copy(
                    rows_v.at[b],
                    out.at[pl.ds(base, _GROWS)],
                    wsems[b]).wait()

            def body(i, carry):
                cps = []
                for b in range(nbuf):
                    @pl.when(i > 0)
                    def _w(b=b):
                        wait_write(b)
                    cps.append(fire(i * nbuf + b, b))
                for b in range(nbuf):
                    drain(i * nbuf + b, b, cps[b])
                return carry

            lax.fori_loop(0, nfull, body, 0)

            tail = []
            for b in range(rem):
                g = nfull * nbuf + b
                wait_write(b)
                tail.append(fire(g, b))
            for b in range(rem):
                drain(nfull * nbuf + b, b, tail[b])
            for b in range(nbuf):
                wait_write(b)

        run_job(in_hbm, inp_iv, inp_groups, 2, o_inp, inp_per_w)
        run_job(out_hbm, pos_iv, ctx_groups, _NBUF, o_pos, ctx_per_w)
        run_job(out_hbm, neg_iv, ctx_groups, _NBUF, o_neg, ctx_per_w)

    return gather_kernel(inp_idx, pos_idx, neg_idx, in_embed, out_embed)


_BLK = 1024
_GRID = _B // _BLK


def _loss_body(pos_ref, neg_ref, inp_ref, w_ref, b_ref, out_ref, acc_ref):
    i = pl.program_id(0)
    a = inp_ref[...]
    an = jnp.sqrt(jnp.sum(a * a, axis=1))

    def proj(ref):
        t = jnp.zeros((_BLK, _D), jnp.float32) + b_ref[...]
        for c in range(_CTX):
            t = t + jnp.dot(ref[c], w_ref[c],
                            preferred_element_type=jnp.float32)
        return t

    def cosv(x):
        num = jnp.sum(a * x, axis=1)
        den = jnp.maximum(an * jnp.sqrt(jnp.sum(x * x, axis=1)), 1e-8)
        return num / den

    c = (1.0 - cosv(proj(pos_ref))) + jnp.maximum(cosv(proj(neg_ref)), 0.0)

    @pl.when(i == 0)
    def _init():
        acc_ref[...] = jnp.zeros_like(acc_ref)

    acc_ref[...] += c.reshape(8, _BLK // 8)

    @pl.when(i == _GRID - 1)
    def _fin():
        loss = jnp.sum(acc_ref[...]) * (1.0 / _B)
        out_ref[...] = loss + jnp.zeros((8, _BLK // 8), jnp.float32)


def _tc_loss(pos3, neg3, inp_rows, w1r, b1):
    return pl.pallas_call(
        _loss_body,
        grid=(_GRID,),
        in_specs=[
            pl.BlockSpec((_CTX, _BLK, _D), lambda i: (0, i, 0)),
            pl.BlockSpec((_CTX, _BLK, _D), lambda i: (0, i, 0)),
            pl.BlockSpec((_BLK, _D), lambda i: (i, 0)),
            pl.BlockSpec((_CTX, _D, _D), lambda i: (0, 0, 0)),
            pl.BlockSpec((1, _D), lambda i: (0, 0)),
        ],
        out_specs=pl.BlockSpec((8, _BLK // 8), lambda i: (0, 0)),
        out_shape=jax.ShapeDtypeStruct((8, _BLK // 8), jnp.float32),
        scratch_shapes=[pltpu.VMEM((8, _BLK // 8), jnp.float32)],
    )(pos3, neg3, inp_rows, w1r, b1)


def kernel(input_labels, pos_labels, neg_labels, in_embed, out_embed, W1, b1):
    nc, ns = _sc_info()
    nw = nc * ns
    inp_idx = input_labels.astype(jnp.int32).reshape(nw, -1, _CHUNK)
    pos_idx = pos_labels.astype(jnp.int32).T.reshape(nw, -1, _CHUNK)
    neg_idx = neg_labels.astype(jnp.int32).T.reshape(nw, -1, _CHUNK)

    inp_rows, pos_rows, neg_rows = _sc_gather(
        inp_idx, pos_idx, neg_idx, in_embed, out_embed, nc, ns)

    w1r = W1.reshape(_D, _CTX, _D).transpose(1, 2, 0)

    acc = _tc_loss(
        pos_rows.reshape(_CTX, _B, _D),
        neg_rows.reshape(_CTX, _B, _D),
        inp_rows,
        w1r,
        b1.reshape(1, _D),
    )
    return acc[0, 0]
